# baseline (speedup 1.0000x reference)
import functools

import jax
import jax.numpy as jnp
from jax import lax
from jax.experimental import pallas as pl
from jax.experimental.pallas import tpu as pltpu
from jax.experimental.pallas import tpu_sc as plsc

NW = 32
NG = 3
NT = 2
PP = 2
XW = 32


def _pos_encoding(length, d_model):
    depth = d_model / 2
    pos = jnp.arange(0, length, dtype=jnp.float32)[:, None]
    i = jnp.arange(0, depth, dtype=jnp.float32)
    angle = pos / jnp.power(10000.0, 2.0 * i / depth)
    return jnp.concatenate([jnp.sin(angle), jnp.cos(angle)], axis=-1)


def kernel(x, W):
    B, L = x.shape
    V, D = W.shape
    BS = B // NW
    S = L // PP
    GR = PP * BS
    peb = jnp.broadcast_to((_pos_encoding(L, D) / 8.0)[:, None, :],
                           (L, BS, D)).reshape(S, GR, D)

    mesh = plsc.VectorSubcoreMesh(core_axis_name="c", subcore_axis_name="s")

    @functools.partial(
        pl.kernel,
        out_type=jax.ShapeDtypeStruct((L, D // 8, B // 128, 8, 128),
                                      jnp.float32),
        mesh=mesh,
        scratch_types=[
            pltpu.VMEM((XW, L), jnp.int32),
            pltpu.VMEM((S, GR), jnp.int32),
            pltpu.VMEM((NG, GR, D), jnp.float32),
            pltpu.VMEM((NT, PP, D // 8, 8, BS), jnp.float32),
            pltpu.SemaphoreType.DMA((NG,)),
            pltpu.SemaphoreType.DMA((NG,)),
            pltpu.SemaphoreType.DMA((NT,)),
        ],
        compiler_params=pltpu.CompilerParams(use_tc_tiling_on_sc=False,
                                             needs_layout_passes=False),
    )
    def sc_run(w_hbm, x_hbm, peb_hbm, out_hbm,
               idxb, idxT, gbuf, tbuf, isem, gsem, osem):
        wid = lax.axis_index("s") * 2 + lax.axis_index("c")
        b0 = wid * BS
        ar = jnp.arange(16, dtype=jnp.int32)

        for p in range(BS // XW):
            pltpu.sync_copy(x_hbm.at[pl.ds(b0 + p * XW, XW)], idxb)

            def ib(l, carry):
                lane = jnp.full((16,), l, dtype=jnp.int32)
                c = lax.div(l, PP)
                colbase = lax.rem(l, PP) * BS + p * XW
                for q in range(XW // 16):
                    idxT[c, pl.ds(colbase + 16 * q, 16)] = plsc.load_gather(
                        idxb, [ar + 16 * q, lane])
                return carry

            lax.fori_loop(0, L, ib, 0)

        def init_start(c, s):
            pltpu.async_copy(peb_hbm.at[c], gbuf.at[s], isem.at[s])

        def init_wait(c, s):
            pltpu.make_async_copy(peb_hbm.at[c], gbuf.at[s],
                                  isem.at[s]).wait()

        def gather_start(c, s):
            pltpu.async_copy(w_hbm.at[idxT.at[c]], gbuf.at[s], gsem.at[s],
                             add=True)

        def gather_wait(c, s):
            pltpu.make_async_copy(w_hbm.at[pl.ds(0, GR)], gbuf.at[s],
                                  gsem.at[s]).wait()

        def out_start(c, t):
            pltpu.async_copy(tbuf.at[t], out_hbm.at[pl.ds(PP * c, PP), :, wid],
                             osem.at[t])

        def out_wait(c, t):
            pltpu.make_async_copy(tbuf.at[t],
                                  out_hbm.at[pl.ds(PP * c, PP), :, wid],
                                  osem.at[t]).wait()

        diag = [lax.rem(ar + j, 16) for j in range(16)]

        def transpose(s, t):
            def tp(g, carry):
                brow = ar + 16 * g
                par = jnp.full((16,), lax.div(g, BS // 16), dtype=jnp.int32)
                bmod = ar + 16 * lax.rem(g, BS // 16)
                for h in range(D // 16):
                    for j in range(16):
                        dcol = diag[j] + 16 * h
                        dh = lax.shift_right_logical(dcol, 3)
                        dl = jnp.bitwise_and(dcol, 7)
                        v = plsc.load_gather(gbuf.at[s], [brow, dcol])
                        plsc.store_scatter(tbuf.at[t], [par, dh, dl, bmod],
                                           v * 8.0)
                return carry

            lax.fori_loop(0, GR // 16, tp, 0)

        init_start(0, 0)

        def body(o, carry):
            for i in range(6):
                c = 6 * o + i
                sg = i % NG
                st = i % NT

                @pl.when(jnp.logical_and(c >= 4, c <= S + 3))
                def _():
                    out_wait(c - 4, st)

                @pl.when(jnp.logical_and(c >= 2, c <= S + 1))
                def _():
                    gather_wait(c - 2, (i + 1) % NG)
                    transpose((i + 1) % NG, st)
                    out_start(c - 2, st)

                @pl.when(c <= S - 1)
                def _():
                    init_wait(c, sg)
                    gather_start(c, sg)

                @pl.when(c <= S - 2)
                def _():
                    init_start(c + 1, (i + 1) % NG)

            return carry

        lax.fori_loop(0, (S + 9) // 6, body, 0)

    out = sc_run(W, x, peb)
    return jnp.transpose(out, (2, 4, 0, 1, 3)).reshape(B, L, D)


# speedup vs baseline: 1.0814x; 1.0814x over previous
import functools

import jax
import jax.numpy as jnp
from jax import lax
from jax.experimental import pallas as pl
from jax.experimental.pallas import tpu as pltpu
from jax.experimental.pallas import tpu_sc as plsc

NW = 32
NG = 4
NT = 2


def _pos_encoding(length, d_model):
    depth = d_model / 2
    pos = jnp.arange(0, length, dtype=jnp.float32)[:, None]
    i = jnp.arange(0, depth, dtype=jnp.float32)
    angle = pos / jnp.power(10000.0, 2.0 * i / depth)
    return jnp.concatenate([jnp.sin(angle), jnp.cos(angle)], axis=-1)


def kernel(x, W):
    B, L = x.shape
    V, D = W.shape
    BS = B // NW
    NC = BS // 4
    peb = jnp.broadcast_to((_pos_encoding(L, D) / 8.0)[:, None, :], (L, NC, D))

    mesh = plsc.VectorSubcoreMesh(core_axis_name="c", subcore_axis_name="s")

    @functools.partial(
        pl.kernel,
        out_type=jax.ShapeDtypeStruct((L, D // 8, B // 128, 8, 128),
                                      jnp.float32),
        mesh=mesh,
        scratch_types=[
            pltpu.VMEM((BS, L), jnp.int32),
            pltpu.VMEM((L, BS), jnp.int32),
            pltpu.VMEM_SHARED((L, NC, D), jnp.float32),
            pltpu.VMEM((NG, BS, D), jnp.float32),
            pltpu.VMEM((NT, D // 8, 8, BS), jnp.float32),
            pltpu.SemaphoreType.DMA((NG,)),
            pltpu.SemaphoreType.DMA((NG,)),
            pltpu.SemaphoreType.DMA((NT,)),
        ],
        compiler_params=pltpu.CompilerParams(use_tc_tiling_on_sc=False,
                                             needs_layout_passes=False),
    )
    def sc_run(w_hbm, x_hbm, peb_hbm, out_hbm,
               idxb, idxT, speb, gbuf, tbuf, isem, gsem, osem):
        wid = lax.axis_index("s") * 2 + lax.axis_index("c")
        b0 = wid * BS
        rows = [jnp.arange(16, dtype=jnp.int32) + 16 * g for g in range(8)]

        pltpu.sync_copy(x_hbm.at[pl.ds(b0, BS)], idxb)

        @pl.when(lax.axis_index("s") == 0)
        def _():
            pltpu.sync_copy(peb_hbm, speb)

        def ib(l, carry):
            lane = jnp.full((16,), l, dtype=jnp.int32)
            for g in range(8):
                idxT[l, pl.ds(16 * g, 16)] = plsc.load_gather(
                    idxb, [rows[g], lane])
            return carry

        lax.fori_loop(0, L, ib, 0)
        plsc.subcore_barrier()

        def init_start(c, s):
            for half in range(4):
                pltpu.async_copy(speb.at[c],
                                 gbuf.at[s, pl.ds(half * NC, NC)],
                                 isem.at[s])

        def init_wait(c, s):
            for half in range(4):
                pltpu.make_async_copy(speb.at[c],
                                      gbuf.at[s, pl.ds(half * NC, NC)],
                                      isem.at[s]).wait()

        def gather_start(c, s):
            pltpu.async_copy(w_hbm.at[idxT.at[c]], gbuf.at[s], gsem.at[s],
                             add=True)

        def gather_wait(c, s):
            pltpu.make_async_copy(w_hbm.at[pl.ds(0, BS)], gbuf.at[s],
                                  gsem.at[s]).wait()

        def out_start(c, t):
            pltpu.async_copy(tbuf.at[t], out_hbm.at[c, :, wid], osem.at[t])

        def out_wait(c, t):
            pltpu.make_async_copy(tbuf.at[t], out_hbm.at[c, :, wid],
                                  osem.at[t]).wait()

        ar = jnp.arange(16, dtype=jnp.int32)
        diag = [lax.rem(ar + j, 16) for j in range(16)]

        def transpose(s, t):
            def tp(g, carry):
                brow = ar + 16 * g
                for h in range(D // 16):
                    for j in range(16):
                        dcol = diag[j] + 16 * h
                        dh = lax.shift_right_logical(dcol, 3)
                        dl = jnp.bitwise_and(dcol, 7)
                        v = plsc.load_gather(gbuf.at[s], [brow, dcol])
                        plsc.store_scatter(tbuf.at[t], [dh, dl, brow],
                                           v * 8.0)
                return carry

            lax.fori_loop(0, BS // 16, tp, 0)

        init_start(0, 0)
        init_start(1, 1)

        def body(o, carry):
            for i in range(NG):
                l = NG * o + i

                @pl.when(jnp.logical_and(l >= NG, l <= L + NG - 1))
                def _():
                    out_wait(l - NG, i % NT)

                @pl.when(jnp.logical_and(l >= 2, l <= L + 1))
                def _():
                    gather_wait(l - 2, (i + 2) % NG)
                    transpose((i + 2) % NG, i % NT)
                    out_start(l - 2, i % NT)

                @pl.when(l <= L - 1)
                def _():
                    init_wait(l, i)
                    gather_start(l, i)

                @pl.when(l <= L - 3)
                def _():
                    init_start(l + 2, (i + 2) % NG)

            return carry

        lax.fori_loop(0, (L + NG) // NG, body, 0)

    out = sc_run(W, x, peb)
    return jnp.transpose(out, (2, 4, 0, 1, 3)).reshape(B, L, D)
